# baseline (speedup 1.0000x reference)
import functools

import jax
import jax.numpy as jnp
from jax import lax
from jax.experimental import pallas as pl
from jax.experimental.pallas import tpu as pltpu
from jax.experimental.pallas import tpu_sc as plsc

_BATCH, _HIST, _DIM = 16384, 50, 32

_info = plsc.get_sparse_core_info()
_NC, _NS = _info.num_cores, _info.num_subcores
_NW = _NC * _NS
_BPW = _BATCH // _NW
_NJ = _BPW // 128
_HSLAB = (_DIM // 8) * (_BATCH // 128) * 8 * 128
_RSLAB = (_BATCH // 128) * 8 * 128

_mesh = plsc.VectorSubcoreMesh(core_axis_name="c", subcore_axis_name="s")


@functools.partial(
    pl.kernel,
    mesh=_mesh,
    out_type=jax.ShapeDtypeStruct((_HIST * _HSLAB,), jnp.float32),
    scratch_types=[
        pltpu.VMEM((_HIST, _BPW), jnp.int32),
        pltpu.VMEM((2 * _NJ, 128, _DIM), jnp.float32),
        pltpu.VMEM((2, _NJ * 128 * _DIM), jnp.float32),
        pltpu.SemaphoreType.DMA,
        pltpu.SemaphoreType.DMA,
    ],
    compiler_params=pltpu.CompilerParams(use_tc_tiling_on_sc=False, needs_layout_passes=False),
)
def _emb_gather(xt_hbm, table_hbm, out_hbm, idx_v, gbuf, tbuf, gsem, osem):
    wid = lax.axis_index("s") * _NC + lax.axis_index("c")
    b0 = wid * _BPW
    pltpu.sync_copy(xt_hbm.at[:, pl.ds(b0, _BPW)], idx_v)

    iota16 = lax.iota(jnp.int32, 16)
    rowidx = [iota16 + 16 * kk for kk in range(8)]

    def step(h, carry):
        b2 = (h - 1) % 2

        @pl.when(h < _HIST)
        def _fire_gathers():
            for j in range(_NJ):
                pltpu.async_copy(
                    table_hbm.at[idx_v.at[h, pl.ds(j * 128, 128)]],
                    gbuf.at[(h % 2) * _NJ + j],
                    gsem,
                )

        @pl.when(h >= 1)
        def _transpose_and_writeout():
            hh = h - 1
            for j in range(_NJ):
                pltpu.make_async_copy(
                    table_hbm.at[pl.ds(0, 128)],
                    gbuf.at[b2 * _NJ + j],
                    gsem,
                ).wait()

            @pl.when(h >= 3)
            def _wait_writeout():
                pltpu.make_async_copy(
                    out_hbm.at[pl.ds(0, _NJ * 128 * _DIM)], tbuf.at[b2], osem
                ).wait()

            def tbody(i, c2):
                blk = gbuf.at[b2 * _NJ + i]
                ibase = i * 1024
                col = c2
                for c in range(32):
                    tb = (c // 8) * (_NJ * 8 * 128) + (c % 8) * 128
                    for kk in range(8):
                        v = plsc.load_gather(blk, [rowidx[kk], col])
                        tbuf[b2, pl.ds(ibase + tb + kk * 16, 16)] = v
                    col = col + 1
                return c2

            lax.fori_loop(0, _NJ, tbody, jnp.zeros((16,), jnp.int32))

            obase = hh * _HSLAB + wid * (_NJ * 1024)
            for r in range(_DIM // 8):
                pltpu.async_copy(
                    tbuf.at[b2, pl.ds(r * (_NJ * 1024), _NJ * 1024)],
                    out_hbm.at[pl.ds(obase + r * _RSLAB, _NJ * 1024)],
                    osem,
                )

        return carry

    lax.fori_loop(0, _HIST + 1, step, 0)
    for b2 in (0, 1):
        pltpu.make_async_copy(
            out_hbm.at[pl.ds(0, _NJ * 128 * _DIM)], tbuf.at[b2], osem
        ).wait()


def kernel(x, embedding):
    out = _emb_gather(x.T, embedding)
    out = out.reshape(_HIST, _DIM // 8, _BATCH // 128, 8, 128)
    return out.transpose(2, 4, 0, 1, 3).reshape(_BATCH, _HIST, _DIM)
